# Initial kernel scaffold; baseline (speedup 1.0000x reference)
import jax
import jax.numpy as jnp
from jax.experimental import pallas as pl


def kernel(x, edge_index, W1, b1, W2, b2):
    raise NotImplementedError("write your pallas kernel here")


# speedup vs baseline: 14.8056x; 14.8056x over previous
import functools

import jax
import jax.numpy as jnp
from jax import lax
from jax.experimental import pallas as pl
from jax.experimental.pallas import tpu as pltpu
from jax.experimental.pallas import tpu_sc as plsc

_N = 10000
_E = 320000
_D = 128
_DH = _D // 2
_NC = 2
_NS = 16
_NW = _NC * _NS
_EPT_DEG = _E // _NW
_EPT = _E // _NS
_CB = 80
_CPT = _EPT // _CB
_NPAD = 10112
_RPT = _NPAD // _NS

_mesh = plsc.VectorSubcoreMesh(core_axis_name="c", subcore_axis_name="s")
_sc_params = pltpu.CompilerParams(needs_layout_passes=False)
_sc_params_lin = pltpu.CompilerParams(
    needs_layout_passes=False, use_tc_tiling_on_sc=False
)


@functools.partial(
    pl.kernel,
    mesh=_mesh,
    out_type=jax.ShapeDtypeStruct((_NW * _NPAD,), jnp.float32),
    scratch_types=[
        pltpu.VMEM((_EPT_DEG,), jnp.int32),
        pltpu.VMEM((_NPAD,), jnp.float32),
    ],
    compiler_params=_sc_params,
)
def _deg_sc(row_hbm, out_hbm, idx_v, hist):
    c = lax.axis_index("c")
    s = lax.axis_index("s")
    wid = c * _NS + s
    z16 = jnp.zeros((16,), jnp.float32)

    def zero_body(i, _):
        hist[pl.ds(i * 16, 16)] = z16
        return 0

    lax.fori_loop(0, _NPAD // 16, zero_body, 0)

    pltpu.sync_copy(row_hbm.at[pl.ds(wid * _EPT_DEG, _EPT_DEG)], idx_v)
    ones = jnp.ones((16,), jnp.float32)

    def hist_body(i, _):
        idx = idx_v[pl.ds(i * 16, 16)]
        plsc.addupdate_scatter(hist, [idx], ones)
        return 0

    lax.fori_loop(0, _EPT_DEG // 16, hist_body, 0)

    pltpu.sync_copy(hist, out_hbm.at[pl.ds(wid * _NPAD, _NPAD)])


@functools.partial(
    pl.kernel,
    mesh=_mesh,
    out_type=jax.ShapeDtypeStruct((_NC * _NPAD, _DH), jnp.float32),
    scratch_types=[
        pltpu.VMEM((_CPT, _CB), jnp.int32),
        pltpu.VMEM((_CPT, _CB), jnp.int32),
        pltpu.VMEM((_CB, _DH), jnp.float32),
        pltpu.VMEM((128, _DH), jnp.float32),
        pltpu.VMEM_SHARED((_NPAD, _DH), jnp.float32),
        pltpu.SemaphoreType.DMA,
    ],
    compiler_params=_sc_params_lin,
)
def _agg_sc(hs_hbm, row2_hbm, col2_hbm, out_hbm, ridx, cidx, rows, zbuf, accum, gsem):
    c = lax.axis_index("c")
    s = lax.axis_index("s")
    z16 = jnp.zeros((16,), jnp.float32)

    def zb(i, _):
        r = i // (_DH // 16)
        q = i % (_DH // 16)
        zbuf[r, pl.ds(q * 16, 16)] = z16
        return 0

    lax.fori_loop(0, 128 * (_DH // 16), zb, 0)

    rbase = s * _RPT
    for k in range(_RPT // 128):
        pltpu.sync_copy(zbuf, accum.at[pl.ds(rbase + k * 128, 128), :])
    rem = _RPT % 128
    if rem:
        pltpu.sync_copy(
            zbuf.at[pl.ds(0, rem), :],
            accum.at[pl.ds(rbase + (_RPT // 128) * 128, rem), :],
        )
    plsc.subcore_barrier()

    pltpu.sync_copy(row2_hbm.at[s], ridx)
    pltpu.sync_copy(col2_hbm.at[s], cidx)

    def chunk(i, _):
        pltpu.async_copy(hs_hbm.at[c].at[ridx.at[i]], rows, gsem).wait()
        pltpu.sync_copy(rows, accum.at[cidx.at[i]], add=True)
        return 0

    lax.fori_loop(0, _CPT, chunk, 0)
    plsc.subcore_barrier()

    pltpu.sync_copy(
        accum.at[pl.ds(rbase, _RPT), :],
        out_hbm.at[pl.ds(c * _NPAD + rbase, _RPT), :],
    )


def _pre_body(x_ref, w_ref, b_ref, dg_ref, hs_ref, dis_ref):
    dp = dg_ref[...]
    deg = jnp.sum(dp, axis=1, keepdims=True) + 1.0
    dis = lax.rsqrt(deg)[0:_N, :]
    h = lax.dot_general(
        x_ref[...], w_ref[...], (((1,), (1,)), ((), ())),
        preferred_element_type=jnp.float32,
    ) + b_ref[...]
    hs = h * dis
    hs_ref[0] = hs[:, 0:_DH]
    hs_ref[1] = hs[:, _DH:_D]
    dis_ref[...] = dis


_pre = pl.pallas_call(
    _pre_body,
    out_shape=(
        jax.ShapeDtypeStruct((_NC, _N, _DH), jnp.float32),
        jax.ShapeDtypeStruct((_N, 1), jnp.float32),
    ),
)


def _mid_body(agg_ref, hs1_ref, dis_ref, w_ref, b_ref, hs2_ref):
    dis = dis_ref[...]
    a = (agg_ref[...] + hs1_ref[...]) * dis
    o1 = jnp.maximum(a, 0.0)
    h1 = jnp.concatenate([o1[0], o1[1]], axis=1)
    h2 = lax.dot_general(
        h1, w_ref[...], (((1,), (1,)), ((), ())),
        preferred_element_type=jnp.float32,
    ) + b_ref[...]
    hs2 = h2 * dis
    hs2_ref[0] = hs2[:, 0:_DH]
    hs2_ref[1] = hs2[:, _DH:_D]


_mid = pl.pallas_call(
    _mid_body,
    out_shape=jax.ShapeDtypeStruct((_NC, _N, _DH), jnp.float32),
)


def _post_body(agg_ref, hs2_ref, dis_ref, out_ref):
    dis = dis_ref[...]
    hsp = (agg_ref[...] + hs2_ref[...]) * dis
    h = jnp.concatenate([hsp[0], hsp[1]], axis=1)
    m = jnp.max(h, axis=1, keepdims=True)
    ex = jnp.exp(h - m)
    se = jnp.sum(ex, axis=1, keepdims=True)
    out_ref[...] = h - m - jnp.log(se)


_post = pl.pallas_call(
    _post_body,
    out_shape=jax.ShapeDtypeStruct((_N, _D), jnp.float32),
)


def kernel(x, edge_index, W1, b1, W2, b2):
    row = edge_index[0]
    col = edge_index[1]
    row2 = row.reshape(_NS, _CPT, _CB)
    col2 = col.reshape(_NS, _CPT, _CB)

    deg_flat = _deg_sc(row)
    dg = deg_flat.reshape(_NW, _NPAD).T

    hs1, dis = _pre(x, W1, b1[None, :], dg)

    agg1 = _agg_sc(hs1, row2, col2).reshape(_NC, _NPAD, _DH)[:, :_N, :]
    hs2 = _mid(agg1, hs1, dis, W2, b2[None, :])

    agg2 = _agg_sc(hs2, row2, col2).reshape(_NC, _NPAD, _DH)[:, :_N, :]
    return _post(agg2, hs2, dis)


# speedup vs baseline: 31.9455x; 2.1577x over previous
import functools

import jax
import jax.numpy as jnp
from jax import lax
from jax.experimental import pallas as pl
from jax.experimental.pallas import tpu as pltpu
from jax.experimental.pallas import tpu_sc as plsc

_N = 10000
_E = 320000
_D = 128
_DH = _D // 2
_NC = 2
_NS = 16
_NW = _NC * _NS
_EPT_DEG = _E // _NW
_EPT = _E // _NS
_CB = 100
_CPT = _EPT // _CB
_R = 5
_G = 4
_NPAD = 10112
_RPT = _NPAD // _NS

_mesh = plsc.VectorSubcoreMesh(core_axis_name="c", subcore_axis_name="s")
_sc_params = pltpu.CompilerParams(needs_layout_passes=False)
_sc_params_lin = pltpu.CompilerParams(
    needs_layout_passes=False, use_tc_tiling_on_sc=False
)


@functools.partial(
    pl.kernel,
    mesh=_mesh,
    out_type=jax.ShapeDtypeStruct((_NW * _NPAD,), jnp.float32),
    scratch_types=[
        pltpu.VMEM((_EPT_DEG,), jnp.int32),
        pltpu.VMEM((_NPAD,), jnp.float32),
    ],
    compiler_params=_sc_params,
)
def _deg_sc(row_hbm, out_hbm, idx_v, hist):
    c = lax.axis_index("c")
    s = lax.axis_index("s")
    wid = c * _NS + s
    z16 = jnp.zeros((16,), jnp.float32)

    def zero_body(i, _):
        hist[pl.ds(i * 16, 16)] = z16
        return 0

    lax.fori_loop(0, _NPAD // 16, zero_body, 0)

    pltpu.sync_copy(row_hbm.at[pl.ds(wid * _EPT_DEG, _EPT_DEG)], idx_v)
    ones = jnp.ones((16,), jnp.float32)

    def hist_body(i, _):
        idx = idx_v[pl.ds(i * 16, 16)]
        plsc.addupdate_scatter(hist, [idx], ones)
        return 0

    lax.fori_loop(0, _EPT_DEG // 16, hist_body, 0)

    pltpu.sync_copy(hist, out_hbm.at[pl.ds(wid * _NPAD, _NPAD)])


@functools.partial(
    pl.kernel,
    mesh=_mesh,
    out_type=jax.ShapeDtypeStruct((_NC * _NPAD, _DH), jnp.float32),
    scratch_types=(
        [
            pltpu.VMEM((_CPT, _CB), jnp.int32),
            pltpu.VMEM((_CPT, _CB), jnp.int32),
            pltpu.VMEM((128, _DH), jnp.float32),
            pltpu.VMEM_SHARED((_NPAD, _DH), jnp.float32),
        ]
        + [pltpu.VMEM((_CB, _DH), jnp.float32) for _ in range(_R)]
        + [pltpu.SemaphoreType.DMA for _ in range(_R)]
    ),
    compiler_params=_sc_params_lin,
)
def _agg_sc(hs_hbm, row2_hbm, col2_hbm, out_hbm, ridx, cidx, zbuf, accum, *bufs):
    rows = bufs[:_R]
    gsems = bufs[_R:2 * _R]
    c = lax.axis_index("c")
    s = lax.axis_index("s")
    z16 = jnp.zeros((16,), jnp.float32)

    def zb(i, _):
        r = i // (_DH // 16)
        q = i % (_DH // 16)
        zbuf[r, pl.ds(q * 16, 16)] = z16
        return 0

    lax.fori_loop(0, 128 * (_DH // 16), zb, 0)

    rbase = s * _RPT
    for k in range(_RPT // 128):
        pltpu.sync_copy(zbuf, accum.at[pl.ds(rbase + k * 128, 128), :])
    rem = _RPT % 128
    if rem:
        pltpu.sync_copy(
            zbuf.at[pl.ds(0, rem), :],
            accum.at[pl.ds(rbase + (_RPT // 128) * 128, rem), :],
        )
    plsc.subcore_barrier()

    pltpu.sync_copy(row2_hbm.at[s], ridx)
    pltpu.sync_copy(col2_hbm.at[s], cidx)

    for p in range(_G):
        pltpu.async_copy(hs_hbm.at[c].at[ridx.at[p]], rows[p], gsems[p])

    def group(g, _):
        for p in range(_R):
            k = g * _R + p
            pf = (p + _G) % _R
            j = k + _G

            @pl.when(j < _CPT)
            def _():
                pltpu.async_copy(hs_hbm.at[c].at[ridx.at[j]], rows[pf], gsems[pf])

            pltpu.make_async_copy(
                hs_hbm.at[c].at[pl.ds(0, _CB), :], rows[p], gsems[p]
            ).wait()
            pltpu.sync_copy(rows[p], accum.at[cidx.at[k]], add=True)
        return 0

    lax.fori_loop(0, _CPT // _R, group, 0)
    plsc.subcore_barrier()

    pltpu.sync_copy(
        accum.at[pl.ds(rbase, _RPT), :],
        out_hbm.at[pl.ds(c * _NPAD + rbase, _RPT), :],
    )


def _pre_body(x_ref, w_ref, b_ref, dg_ref, hs_ref, dis_ref):
    dp = dg_ref[...]
    deg = jnp.sum(dp, axis=1, keepdims=True) + 1.0
    dis = lax.rsqrt(deg)[0:_N, :]
    h = lax.dot_general(
        x_ref[...], w_ref[...], (((1,), (1,)), ((), ())),
        preferred_element_type=jnp.float32,
    ) + b_ref[...]
    hs = h * dis
    hs_ref[0] = hs[:, 0:_DH]
    hs_ref[1] = hs[:, _DH:_D]
    dis_ref[...] = dis


_pre = pl.pallas_call(
    _pre_body,
    out_shape=(
        jax.ShapeDtypeStruct((_NC, _N, _DH), jnp.float32),
        jax.ShapeDtypeStruct((_N, 1), jnp.float32),
    ),
)


def _mid_body(agg_ref, hs1_ref, dis_ref, w_ref, b_ref, hs2_ref):
    dis = dis_ref[...]
    a = (agg_ref[...] + hs1_ref[...]) * dis
    o1 = jnp.maximum(a, 0.0)
    h1 = jnp.concatenate([o1[0], o1[1]], axis=1)
    h2 = lax.dot_general(
        h1, w_ref[...], (((1,), (1,)), ((), ())),
        preferred_element_type=jnp.float32,
    ) + b_ref[...]
    hs2 = h2 * dis
    hs2_ref[0] = hs2[:, 0:_DH]
    hs2_ref[1] = hs2[:, _DH:_D]


_mid = pl.pallas_call(
    _mid_body,
    out_shape=jax.ShapeDtypeStruct((_NC, _N, _DH), jnp.float32),
)


def _post_body(agg_ref, hs2_ref, dis_ref, out_ref):
    dis = dis_ref[...]
    hsp = (agg_ref[...] + hs2_ref[...]) * dis
    h = jnp.concatenate([hsp[0], hsp[1]], axis=1)
    m = jnp.max(h, axis=1, keepdims=True)
    ex = jnp.exp(h - m)
    se = jnp.sum(ex, axis=1, keepdims=True)
    out_ref[...] = h - m - jnp.log(se)


_post = pl.pallas_call(
    _post_body,
    out_shape=jax.ShapeDtypeStruct((_N, _D), jnp.float32),
)


def kernel(x, edge_index, W1, b1, W2, b2):
    row = edge_index[0]
    col = edge_index[1]
    row2 = row.reshape(_NS, _CPT, _CB)
    col2 = col.reshape(_NS, _CPT, _CB)

    deg_flat = _deg_sc(row)
    dg = deg_flat.reshape(_NW, _NPAD).T

    hs1, dis = _pre(x, W1, b1[None, :], dg)

    agg1 = _agg_sc(hs1, row2, col2).reshape(_NC, _NPAD, _DH)[:, :_N, :]
    hs2 = _mid(agg1, hs1, dis, W2, b2[None, :])

    agg2 = _agg_sc(hs2, row2, col2).reshape(_NC, _NPAD, _DH)[:, :_N, :]
    return _post(agg2, hs2, dis)


# speedup vs baseline: 33.6650x; 1.0538x over previous
import functools

import jax
import jax.numpy as jnp
from jax import lax
from jax.experimental import pallas as pl
from jax.experimental.pallas import tpu as pltpu
from jax.experimental.pallas import tpu_sc as plsc

_N = 10000
_E = 320000
_D = 128
_DH = _D // 2
_NC = 2
_NS = 16
_NW = _NC * _NS
_EPT_DEG = _E // _NW
_EPT = _E // _NS
_CB = 100
_CPT = _EPT // _CB
_R = 5
_G = 4
_NPAD = 10112
_RPT = _NPAD // _NS

_mesh = plsc.VectorSubcoreMesh(core_axis_name="c", subcore_axis_name="s")
_sc_params = pltpu.CompilerParams(needs_layout_passes=False)
_sc_params_lin = pltpu.CompilerParams(
    needs_layout_passes=False, use_tc_tiling_on_sc=False
)


@functools.partial(
    pl.kernel,
    mesh=_mesh,
    out_type=jax.ShapeDtypeStruct((_NW * _NPAD,), jnp.float32),
    scratch_types=[
        pltpu.VMEM((_EPT_DEG,), jnp.int32),
        pltpu.VMEM((_NPAD,), jnp.float32),
    ],
    compiler_params=_sc_params,
)
def _deg_sc(row_hbm, out_hbm, idx_v, hist):
    c = lax.axis_index("c")
    s = lax.axis_index("s")
    wid = c * _NS + s
    z16 = jnp.zeros((16,), jnp.float32)

    def zero_body(i, _):
        hist[pl.ds(i * 16, 16)] = z16
        return 0

    lax.fori_loop(0, _NPAD // 16, zero_body, 0)

    pltpu.sync_copy(row_hbm.at[pl.ds(wid * _EPT_DEG, _EPT_DEG)], idx_v)
    ones = jnp.ones((16,), jnp.float32)

    def hist_body(i, _):
        idx = idx_v[pl.ds(i * 16, 16)]
        plsc.addupdate_scatter(hist, [idx], ones)
        return 0

    lax.fori_loop(0, _EPT_DEG // 16, hist_body, 0)

    pltpu.sync_copy(hist, out_hbm.at[pl.ds(wid * _NPAD, _NPAD)])


@functools.partial(
    pl.kernel,
    mesh=_mesh,
    out_type=jax.ShapeDtypeStruct((_NC * _NPAD, _DH), jnp.float32),
    scratch_types=(
        [
            pltpu.VMEM((_CPT, _CB), jnp.int32),
            pltpu.VMEM((_CPT, _CB), jnp.int32),
            pltpu.VMEM((128, _DH), jnp.float32),
            pltpu.VMEM_SHARED((_NPAD, _DH), jnp.float32),
        ]
        + [pltpu.VMEM((_CB, _DH), jnp.float32) for _ in range(_R)]
        + [pltpu.SemaphoreType.DMA for _ in range(_R)]
    ),
    compiler_params=_sc_params_lin,
)
def _agg_sc(hs_hbm, row2_hbm, col2_hbm, out_hbm, ridx, cidx, zbuf, accum, *bufs):
    rows = bufs[:_R]
    gsems = bufs[_R:2 * _R]
    c = lax.axis_index("c")
    s = lax.axis_index("s")
    z16 = jnp.zeros((16,), jnp.float32)

    def zb(i, _):
        r = i // (_DH // 16)
        q = i % (_DH // 16)
        zbuf[r, pl.ds(q * 16, 16)] = z16
        return 0

    lax.fori_loop(0, 128 * (_DH // 16), zb, 0)

    rbase = s * _RPT
    for k in range(_RPT // 128):
        pltpu.sync_copy(zbuf, accum.at[pl.ds(rbase + k * 128, 128), :])
    rem = _RPT % 128
    if rem:
        pltpu.sync_copy(
            zbuf.at[pl.ds(0, rem), :],
            accum.at[pl.ds(rbase + (_RPT // 128) * 128, rem), :],
        )
    plsc.subcore_barrier()

    pltpu.sync_copy(row2_hbm.at[s], ridx)
    pltpu.sync_copy(col2_hbm.at[s], cidx)

    for p in range(_G):
        pltpu.async_copy(hs_hbm.at[c].at[ridx.at[p]], rows[p], gsems[p])

    def group(g, _):
        for p in range(_R):
            k = g * _R + p
            pf = (p + _G) % _R
            j = k + _G

            @pl.when(j < _CPT)
            def _():
                pltpu.async_copy(hs_hbm.at[c].at[ridx.at[j]], rows[pf], gsems[pf])

            pltpu.make_async_copy(
                hs_hbm.at[c].at[pl.ds(0, _CB), :], rows[p], gsems[p]
            ).wait()
            pltpu.sync_copy(rows[p], accum.at[cidx.at[k]], add=True)
        return 0

    lax.fori_loop(0, _CPT // _R, group, 0)
    plsc.subcore_barrier()

    pltpu.sync_copy(
        accum.at[pl.ds(rbase, _RPT), :],
        out_hbm.at[pl.ds(c * _NPAD + rbase, _RPT), :],
    )


def _pre_body(x_ref, w_ref, b_ref, dg_ref, hs_ref, dis_ref):
    dp = dg_ref[...]
    ones_col = jnp.ones((_NW, 1), jnp.float32)
    deg = lax.dot_general(
        dp, ones_col, (((0,), (0,)), ((), ())),
        preferred_element_type=jnp.float32,
    ) + 1.0
    dis = lax.rsqrt(deg)[0:_N, :]
    h = lax.dot_general(
        x_ref[...], w_ref[...], (((1,), (1,)), ((), ())),
        preferred_element_type=jnp.float32,
    ) + b_ref[...]
    hs = h * dis
    hs_ref[0] = hs[:, 0:_DH]
    hs_ref[1] = hs[:, _DH:_D]
    dis_ref[...] = dis


_pre = pl.pallas_call(
    _pre_body,
    out_shape=(
        jax.ShapeDtypeStruct((_NC, _N, _DH), jnp.float32),
        jax.ShapeDtypeStruct((_N, 1), jnp.float32),
    ),
)


def _mid_body(agg_ref, hs1_ref, dis_ref, w_ref, b_ref, hs2_ref):
    dis = dis_ref[...]
    a = (agg_ref[:, 0:_N, :] + hs1_ref[...]) * dis
    o1 = jnp.maximum(a, 0.0)
    h1 = jnp.concatenate([o1[0], o1[1]], axis=1)
    h2 = lax.dot_general(
        h1, w_ref[...], (((1,), (1,)), ((), ())),
        preferred_element_type=jnp.float32,
    ) + b_ref[...]
    hs2 = h2 * dis
    hs2_ref[0] = hs2[:, 0:_DH]
    hs2_ref[1] = hs2[:, _DH:_D]


_mid = pl.pallas_call(
    _mid_body,
    out_shape=jax.ShapeDtypeStruct((_NC, _N, _DH), jnp.float32),
)


def _post_body(agg_ref, hs2_ref, dis_ref, out_ref):
    dis = dis_ref[...]
    hsp = (agg_ref[:, 0:_N, :] + hs2_ref[...]) * dis
    h = jnp.concatenate([hsp[0], hsp[1]], axis=1)
    m = jnp.max(h, axis=1, keepdims=True)
    ex = jnp.exp(h - m)
    se = jnp.sum(ex, axis=1, keepdims=True)
    out_ref[...] = h - m - jnp.log(se)


_post = pl.pallas_call(
    _post_body,
    out_shape=jax.ShapeDtypeStruct((_N, _D), jnp.float32),
)


def kernel(x, edge_index, W1, b1, W2, b2):
    row = edge_index[0]
    col = edge_index[1]
    row2 = row.reshape(_NS, _CPT, _CB)
    col2 = col.reshape(_NS, _CPT, _CB)

    deg_parts = _deg_sc(row).reshape(_NW, _NPAD)

    hs1, dis = _pre(x, W1, b1[None, :], deg_parts)

    agg1 = _agg_sc(hs1, row2, col2).reshape(_NC, _NPAD, _DH)
    hs2 = _mid(agg1, hs1, dis, W2, b2[None, :])

    agg2 = _agg_sc(hs2, row2, col2).reshape(_NC, _NPAD, _DH)
    return _post(agg2, hs2, dis)


# speedup vs baseline: 33.9297x; 1.0079x over previous
import functools

import jax
import jax.numpy as jnp
from jax import lax
from jax.experimental import pallas as pl
from jax.experimental.pallas import tpu as pltpu
from jax.experimental.pallas import tpu_sc as plsc

_N = 10000
_E = 320000
_D = 128
_DH = _D // 2
_NC = 2
_NS = 16
_NW = _NC * _NS
_EPT_DEG = _E // _NW
_EPT = _E // _NS
_CB = 125
_CPT = _EPT // _CB
_R = 4
_G = 3
_NPAD = 10112
_RPT = _NPAD // _NS

_mesh = plsc.VectorSubcoreMesh(core_axis_name="c", subcore_axis_name="s")
_sc_params = pltpu.CompilerParams(needs_layout_passes=False)
_sc_params_lin = pltpu.CompilerParams(
    needs_layout_passes=False, use_tc_tiling_on_sc=False
)


@functools.partial(
    pl.kernel,
    mesh=_mesh,
    out_type=jax.ShapeDtypeStruct((_NW * _NPAD,), jnp.float32),
    scratch_types=[
        pltpu.VMEM((_EPT_DEG,), jnp.int32),
        pltpu.VMEM((_NPAD,), jnp.float32),
    ],
    compiler_params=_sc_params,
)
def _deg_sc(row_hbm, out_hbm, idx_v, hist):
    c = lax.axis_index("c")
    s = lax.axis_index("s")
    wid = c * _NS + s
    z16 = jnp.zeros((16,), jnp.float32)

    def zero_body(i, _):
        hist[pl.ds(i * 16, 16)] = z16
        return 0

    lax.fori_loop(0, _NPAD // 16, zero_body, 0)

    pltpu.sync_copy(row_hbm.at[pl.ds(wid * _EPT_DEG, _EPT_DEG)], idx_v)
    ones = jnp.ones((16,), jnp.float32)

    def hist_body(i, _):
        idx = idx_v[pl.ds(i * 16, 16)]
        plsc.addupdate_scatter(hist, [idx], ones)
        return 0

    lax.fori_loop(0, _EPT_DEG // 16, hist_body, 0)

    pltpu.sync_copy(hist, out_hbm.at[pl.ds(wid * _NPAD, _NPAD)])


@functools.partial(
    pl.kernel,
    mesh=_mesh,
    out_type=jax.ShapeDtypeStruct((_NC * _NPAD, _DH), jnp.float32),
    scratch_types=(
        [
            pltpu.VMEM((_CPT, _CB), jnp.int32),
            pltpu.VMEM((_CPT, _CB), jnp.int32),
            pltpu.VMEM((128, _DH), jnp.float32),
            pltpu.VMEM_SHARED((_NPAD, _DH), jnp.float32),
        ]
        + [pltpu.VMEM((_CB, _DH), jnp.float32) for _ in range(_R)]
        + [pltpu.SemaphoreType.DMA for _ in range(_R)]
    ),
    compiler_params=_sc_params_lin,
)
def _agg_sc(hs_hbm, row2_hbm, col2_hbm, out_hbm, ridx, cidx, zbuf, accum, *bufs):
    rows = bufs[:_R]
    gsems = bufs[_R:2 * _R]
    c = lax.axis_index("c")
    s = lax.axis_index("s")
    z16 = jnp.zeros((16,), jnp.float32)

    def zb(i, _):
        r = i // (_DH // 16)
        q = i % (_DH // 16)
        zbuf[r, pl.ds(q * 16, 16)] = z16
        return 0

    lax.fori_loop(0, 128 * (_DH // 16), zb, 0)

    rbase = s * _RPT
    for k in range(_RPT // 128):
        pltpu.sync_copy(zbuf, accum.at[pl.ds(rbase + k * 128, 128), :])
    rem = _RPT % 128
    if rem:
        pltpu.sync_copy(
            zbuf.at[pl.ds(0, rem), :],
            accum.at[pl.ds(rbase + (_RPT // 128) * 128, rem), :],
        )
    plsc.subcore_barrier()

    pltpu.sync_copy(row2_hbm.at[s], ridx)
    pltpu.sync_copy(col2_hbm.at[s], cidx)

    for p in range(_G):
        pltpu.async_copy(hs_hbm.at[c].at[ridx.at[p]], rows[p], gsems[p])

    def group(g, _):
        for p in range(_R):
            k = g * _R + p
            pf = (p + _G) % _R
            j = k + _G

            @pl.when(j < _CPT)
            def _():
                pltpu.async_copy(hs_hbm.at[c].at[ridx.at[j]], rows[pf], gsems[pf])

            pltpu.make_async_copy(
                hs_hbm.at[c].at[pl.ds(0, _CB), :], rows[p], gsems[p]
            ).wait()
            pltpu.sync_copy(rows[p], accum.at[cidx.at[k]], add=True)
        return 0

    lax.fori_loop(0, _CPT // _R, group, 0)
    plsc.subcore_barrier()

    pltpu.sync_copy(
        accum.at[pl.ds(rbase, _RPT), :],
        out_hbm.at[pl.ds(c * _NPAD + rbase, _RPT), :],
    )


def _pre_body(x_ref, w_ref, b_ref, dg_ref, hs_ref, dis_ref):
    dp = dg_ref[...]
    ones_col = jnp.ones((_NW, 1), jnp.float32)
    deg = lax.dot_general(
        dp, ones_col, (((0,), (0,)), ((), ())),
        preferred_element_type=jnp.float32,
    ) + 1.0
    dis = lax.rsqrt(deg)[0:_N, :]
    h = lax.dot_general(
        x_ref[...], w_ref[...], (((1,), (1,)), ((), ())),
        preferred_element_type=jnp.float32,
    ) + b_ref[...]
    hs = h * dis
    hs_ref[0] = hs[:, 0:_DH]
    hs_ref[1] = hs[:, _DH:_D]
    dis_ref[...] = dis


_pre = pl.pallas_call(
    _pre_body,
    out_shape=(
        jax.ShapeDtypeStruct((_NC, _N, _DH), jnp.float32),
        jax.ShapeDtypeStruct((_N, 1), jnp.float32),
    ),
)


def _mid_body(agg_ref, hs1_ref, dis_ref, w_ref, b_ref, hs2_ref):
    dis = dis_ref[...]
    a = (agg_ref[:, 0:_N, :] + hs1_ref[...]) * dis
    o1 = jnp.maximum(a, 0.0)
    h1 = jnp.concatenate([o1[0], o1[1]], axis=1)
    h2 = lax.dot_general(
        h1, w_ref[...], (((1,), (1,)), ((), ())),
        preferred_element_type=jnp.float32,
    ) + b_ref[...]
    hs2 = h2 * dis
    hs2_ref[0] = hs2[:, 0:_DH]
    hs2_ref[1] = hs2[:, _DH:_D]


_mid = pl.pallas_call(
    _mid_body,
    out_shape=jax.ShapeDtypeStruct((_NC, _N, _DH), jnp.float32),
)


def _post_body(agg_ref, hs2_ref, dis_ref, out_ref):
    dis = dis_ref[...]
    hsp = (agg_ref[:, 0:_N, :] + hs2_ref[...]) * dis
    h = jnp.concatenate([hsp[0], hsp[1]], axis=1)
    m = jnp.max(h, axis=1, keepdims=True)
    ex = jnp.exp(h - m)
    se = jnp.sum(ex, axis=1, keepdims=True)
    out_ref[...] = h - m - jnp.log(se)


_post = pl.pallas_call(
    _post_body,
    out_shape=jax.ShapeDtypeStruct((_N, _D), jnp.float32),
)


def kernel(x, edge_index, W1, b1, W2, b2):
    row = edge_index[0]
    col = edge_index[1]
    row2 = row.reshape(_NS, _CPT, _CB)
    col2 = col.reshape(_NS, _CPT, _CB)

    deg_parts = _deg_sc(row).reshape(_NW, _NPAD)

    hs1, dis = _pre(x, W1, b1[None, :], deg_parts)

    agg1 = _agg_sc(hs1, row2, col2).reshape(_NC, _NPAD, _DH)
    hs2 = _mid(agg1, hs1, dis, W2, b2[None, :])

    agg2 = _agg_sc(hs2, row2, col2).reshape(_NC, _NPAD, _DH)
    return _post(agg2, hs2, dis)
